# baseline (speedup 1.0000x reference)
import functools

import jax
import jax.numpy as jnp
from jax import lax
from jax.experimental import pallas as pl
from jax.experimental.pallas import tpu as pltpu
from jax.experimental.pallas import tpu_sc as plsc

_EPS = 1e-08
_ROWS = 2560
_HW = 4096
_NW = 32
_RPW = _ROWS // _NW


def _sc_select_body(xi_hbm, kp1_hbm, out_hbm, rowvec, mrow, hist, c16,
                    kp1_v, selv, seli):
    wid = lax.axis_index("s") * 2 + lax.axis_index("c")
    base = wid * _RPW
    pltpu.sync_copy(kp1_hbm.at[pl.ds(base, _RPW)], kp1_v)

    lane = lax.iota(jnp.int32, 16)
    ones = jnp.ones((16,), jnp.int32)
    zeros16 = jnp.zeros((16,), jnp.int32)

    def zero_hist():
        @plsc.parallel_loop(0, 256, unroll=16)
        def _(i):
            hist[i] = zeros16
        @plsc.parallel_loop(0, 16, unroll=16)
        def _(i):
            c16[i] = zeros16

    def scan_hist(target):
        @plsc.parallel_loop(0, 16, unroll=4, carry=(jnp.int32(0), jnp.int32(0), jnp.int32(0)))
        def coarse(gi, carry):
            acc, gsel, gabove = carry
            g = 15 - gi
            gsum = jnp.sum(c16[g])
            new_acc = acc + gsum
            hit = (acc < target) & (new_acc >= target)
            gsel = jnp.where(hit, g, gsel)
            gabove = jnp.where(hit, acc, gabove)
            return new_acc, gsel, gabove
        _, gsel, gabove = coarse

        t2 = target - gabove
        @plsc.parallel_loop(0, 16, unroll=4, carry=(jnp.int32(0), jnp.int32(0), jnp.int32(0)))
        def fine(bi, carry):
            acc, bsel, babove = carry
            b = gsel * 16 + (15 - bi)
            bsum = jnp.sum(hist[b])
            new_acc = acc + bsum
            hit = (acc < t2) & (new_acc >= t2)
            bsel = jnp.where(hit, b, bsel)
            babove = jnp.where(hit, acc, babove)
            return new_acc, bsel, babove
        _, bsel, babove = fine
        return bsel, gabove + babove

    def select_row(row, kp1):
        pltpu.sync_copy(xi_hbm.at[row], rowvec)

        zero_hist()
        @plsc.parallel_loop(0, 256, unroll=8)
        def _(c):
            v = rowvec[pl.ds(c * 16, 16)]
            vu = plsc.bitcast(v, jnp.uint32)
            m = jnp.where(v < 0, ~vu, vu | jnp.uint32(0x80000000))
            mrow[pl.ds(c * 16, 16)] = m
            dig = (m >> 24).astype(jnp.int32)
            plsc.addupdate_scatter(hist, [dig, lane], ones)
            plsc.addupdate_scatter(c16, [dig >> 4, lane], ones)
        bin0, above = scan_hist(kp1)
        key = bin0.astype(jnp.uint32) << 24
        target = kp1 - above

        def radix_pass(shift, key, target):
            zero_hist()
            hb = key >> (shift + 8)
            @plsc.parallel_loop(0, 256, unroll=8)
            def _(c):
                m = mrow[pl.ds(c * 16, 16)]
                dig = ((m >> shift) & jnp.uint32(0xFF)).astype(jnp.int32)
                ok = (m >> (shift + 8)) == hb
                plsc.addupdate_scatter(hist, [dig, lane], ones, mask=ok)
                plsc.addupdate_scatter(c16, [dig >> 4, lane], ones, mask=ok)
            b, above = scan_hist(target)
            return key | (b.astype(jnp.uint32) << shift), target - above

        key, target = radix_pass(16, key, target)
        key, target = radix_pass(8, key, target)
        key, _ = radix_pass(0, key, target)
        return key

    def row_body(ri, _):
        g16 = (ri // 16) * 16
        kv = kp1_v[pl.ds(g16, 16)]
        kp1 = jnp.sum(jnp.where(lane == ri - g16, kv, 0))
        key = select_row(base + ri, kp1)
        keyv = plsc.bitcast(jnp.full((16,), key, jnp.uint32), jnp.int32)
        plsc.store_scatter(selv, [jnp.full((16,), ri, jnp.int32)], keyv,
                           mask=lane == 0)
        return 0

    lax.fori_loop(0, _RPW, row_body, 0)

    def unmap(c, _):
        m = plsc.bitcast(selv[pl.ds(c * 16, 16)], jnp.uint32)
        pos = (m & jnp.uint32(0x80000000)) != 0
        xb = jnp.where(pos, m ^ jnp.uint32(0x80000000), ~m)
        seli[pl.ds(c * 16, 16)] = plsc.bitcast(xb, jnp.int32)
        return 0
    lax.fori_loop(0, _RPW // 16, unmap, 0)
    pltpu.sync_copy(seli, out_hbm.at[pl.ds(base, _RPW)])


_sc_select = functools.partial(
    pl.kernel,
    out_type=jax.ShapeDtypeStruct((_ROWS,), jnp.int32),
    compiler_params=pltpu.CompilerParams(needs_layout_passes=False),
    mesh=plsc.VectorSubcoreMesh(core_axis_name="c", subcore_axis_name="s"),
    scratch_types=[
        pltpu.VMEM((_HW,), jnp.int32),
        pltpu.VMEM((_HW,), jnp.uint32),
        pltpu.VMEM((256, 16), jnp.int32),
        pltpu.VMEM((16, 16), jnp.int32),
        pltpu.VMEM((_RPW,), jnp.int32),
        pltpu.VMEM((_RPW,), jnp.int32),
        pltpu.VMEM((_RPW,), jnp.int32),
    ],
)(_sc_select_body)


def _loss_kernel(x_ref, y_ref, sel_ref, out_ref, *, n_total):
    i = pl.program_id(0)
    nb = pl.num_programs(0)
    blk = x_ref.shape[0]

    s = jax.nn.sigmoid(x_ref[...])
    y = y_ref[0, 0, :]
    xk = jax.lax.bitcast_convert_type(sel_ref[0, 0, :], jnp.float32)
    t = jnp.maximum(jax.nn.sigmoid(xk), 1e-4)[:, None]
    yb = y[:, None]

    u = s * (1.0 / t)
    h_low = u * (2.0 - u)
    one_m_t = 1.0 - t
    alpha = 1.0 / jnp.maximum(one_m_t * one_m_t, _EPS)
    h_high = alpha * (1.0 - s) * (one_m_t + (s - t))
    piece = jnp.where(s <= t, h_low, h_high) * yb
    neg = -(1.0 - yb) * jnp.log(jnp.maximum(1.0 - s, _EPS))
    bsum = jnp.sum(piece + neg)

    prev = jnp.where(i == 0, jnp.zeros((1, 1), jnp.float32), out_ref[...])
    acc = prev + bsum
    out_ref[...] = jnp.where(i == nb - 1, acc / n_total, acc)


@functools.partial(jax.jit, static_argnames=("interpret",))
def _spatial_bce(x, y, fg, interpret=False):
    b, c, h, w = x.shape
    hw = h * w
    rows = b * c
    blk = 256
    nb = rows // blk
    x2 = x.reshape(rows, hw)
    xi = jax.lax.bitcast_convert_type(x2, jnp.int32)
    kp1 = (fg.reshape(-1) * hw).astype(jnp.int32) + 1
    sel = _sc_select(xi, kp1)
    y3 = y.reshape(nb, 1, blk)
    sel3 = sel.reshape(nb, 1, blk)
    out = pl.pallas_call(
        functools.partial(_loss_kernel, n_total=rows * hw),
        grid=(nb,),
        in_specs=[
            pl.BlockSpec((blk, hw), lambda i: (i, 0)),
            pl.BlockSpec((1, 1, blk), lambda i: (i, 0, 0)),
            pl.BlockSpec((1, 1, blk), lambda i: (i, 0, 0)),
        ],
        out_specs=pl.BlockSpec((1, 1), lambda i: (0, 0)),
        out_shape=jax.ShapeDtypeStruct((1, 1), jnp.float32),
        interpret=interpret,
    )(x2, y3, sel3)
    return out[0, 0]


def kernel(x, y, fg, iter):
    return _spatial_bce(x, y, fg) + jnp.asarray(iter, jnp.float32) * 0.0


# speedup vs baseline: 1.5986x; 1.5986x over previous
import functools

import jax
import jax.numpy as jnp
from jax.experimental import pallas as pl

_EPS = 1e-08


def _block_kernel(x_ref, y_ref, fg_ref, out_ref, *, n_total):
    i = pl.program_id(0)
    nb = pl.num_programs(0)
    hw = x_ref.shape[1]
    blk = x_ref.shape[0]

    s = jax.nn.sigmoid(x_ref[...])
    si = jax.lax.bitcast_convert_type(s, jnp.int32)

    fg = fg_ref[0, 0, :]
    y = y_ref[0, 0, :]
    kp1 = (fg * hw).astype(jnp.int32) + 1

    def count_ge(data16, cand16b):
        m = (data16 >= cand16b).astype(jnp.int16)
        n = m.shape[1]
        while n > 128:
            n //= 2
            m = m[:, :n] + m[:, n:]
        return jnp.sum(m.astype(jnp.int32), axis=1)

    def search15(data16, target):
        def step(_, carry):
            r, bit = carry
            cand = r | bit
            cnt = count_ge(data16, cand.astype(jnp.int16)[:, None])
            return jnp.where(cnt >= target, cand, r), bit >> 1
        r0 = jnp.zeros((blk,), jnp.int32)
        bit0 = jnp.full((blk,), 1 << 14, jnp.int32)
        r, _ = jax.lax.fori_loop(0, 15, step, (r0, bit0))
        return r

    sh16 = (si >> 15).astype(jnp.int16)
    rh = search15(sh16, kp1)

    rh16b = rh.astype(jnp.int16)[:, None]
    c_gt = count_ge(sh16, rh16b + jnp.int16(1))
    sl16 = jnp.where(sh16 == rh16b,
                     (si & 0x7FFF).astype(jnp.int16), jnp.int16(-1))
    rl = search15(sl16, kp1 - c_gt)

    r = (rh << 15) | rl
    t = jax.lax.bitcast_convert_type(r, jnp.float32)
    t = jnp.maximum(t, 1e-4)[:, None]
    yb = y[:, None]

    u = s * (1.0 / t)
    h_low = u * (2.0 - u)
    one_m_t = 1.0 - t
    alpha = 1.0 / jnp.maximum(one_m_t * one_m_t, _EPS)
    h_high = alpha * (1.0 - s) * (one_m_t + (s - t))
    piece = jnp.where(s <= t, h_low, h_high) * yb
    neg = -(1.0 - yb) * jnp.log(jnp.maximum(1.0 - s, _EPS))
    bsum = jnp.sum(piece + neg)

    prev = jnp.where(i == 0, jnp.zeros((1, 1), jnp.float32), out_ref[...])
    acc = prev + bsum
    out_ref[...] = jnp.where(i == nb - 1, acc / n_total, acc)


@functools.partial(jax.jit, static_argnames=("interpret",))
def _spatial_bce(x, y, fg, interpret=False):
    b, c, h, w = x.shape
    hw = h * w
    rows = b * c
    blk = 256
    nb = rows // blk
    x2 = x.reshape(rows, hw)
    y3 = y.reshape(nb, 1, blk)
    fg3 = fg.reshape(nb, 1, blk)
    out = pl.pallas_call(
        functools.partial(_block_kernel, n_total=rows * hw),
        grid=(nb,),
        in_specs=[
            pl.BlockSpec((blk, hw), lambda i: (i, 0)),
            pl.BlockSpec((1, 1, blk), lambda i: (i, 0, 0)),
            pl.BlockSpec((1, 1, blk), lambda i: (i, 0, 0)),
        ],
        out_specs=pl.BlockSpec((1, 1), lambda i: (0, 0)),
        out_shape=jax.ShapeDtypeStruct((1, 1), jnp.float32),
        interpret=interpret,
    )(x2, y3, fg3)
    return out[0, 0]


def kernel(x, y, fg, iter):
    return _spatial_bce(x, y, fg) + jnp.asarray(iter, jnp.float32) * 0.0


# speedup vs baseline: 1.7339x; 1.0847x over previous
import functools

import jax
import jax.numpy as jnp
from jax.experimental import pallas as pl

_EPS = 1e-08


def _block_kernel(x_ref, y_ref, fg_ref, out_ref, *, n_total):
    i = pl.program_id(0)
    nb = pl.num_programs(0)
    hw = x_ref.shape[1]
    blk = x_ref.shape[0]

    s = jax.nn.sigmoid(x_ref[...])
    si = jax.lax.bitcast_convert_type(s, jnp.int32)

    fg = fg_ref[0, 0, :]
    y = y_ref[0, 0, :]
    kp1 = (fg * hw).astype(jnp.int32) + 1

    def count_ge(data16, cand16b):
        m = (data16 >= cand16b).astype(jnp.int16)
        n = m.shape[1]
        while n > 128:
            n //= 2
            m = m[:, :n] + m[:, n:]
        return jnp.sum(m.astype(jnp.int32), axis=1)

    def search15(data16, target):
        def step(_, carry):
            r, bit = carry
            cand = r | bit
            cnt = count_ge(data16, cand.astype(jnp.int16)[:, None])
            return jnp.where(cnt >= target, cand, r), bit >> 1
        r0 = jnp.zeros((blk,), jnp.int32)
        bit0 = jnp.full((blk,), 1 << 14, jnp.int32)
        r, _ = jax.lax.fori_loop(0, 15, step, (r0, bit0))
        return r

    sh16 = (si >> 15).astype(jnp.int16)
    rh = search15(sh16, kp1)

    rh16b = rh.astype(jnp.int16)[:, None]
    c_gt = count_ge(sh16, rh16b + jnp.int16(1))
    sl16 = jnp.where(sh16 == rh16b,
                     (si & 0x7FFF).astype(jnp.int16), jnp.int16(-1))
    rl = search15(sl16, kp1 - c_gt)

    r = (rh << 15) | rl
    t = jax.lax.bitcast_convert_type(r, jnp.float32)
    t = jnp.maximum(t, 1e-4)[:, None]
    yb = y[:, None]

    u = s * (1.0 / t)
    h_low = u * (2.0 - u)
    one_m_t = 1.0 - t
    alpha = 1.0 / jnp.maximum(one_m_t * one_m_t, _EPS)
    h_high = alpha * (1.0 - s) * (one_m_t + (s - t))
    piece = jnp.where(s <= t, h_low, h_high) * yb
    neg = -(1.0 - yb) * jnp.log(jnp.maximum(1.0 - s, _EPS))
    bsum = jnp.sum(piece + neg)

    prev = jnp.where(i == 0, jnp.zeros((1, 1), jnp.float32), out_ref[...])
    acc = prev + bsum
    out_ref[...] = jnp.where(i == nb - 1, acc / n_total, acc)


@functools.partial(jax.jit, static_argnames=("interpret",))
def _spatial_bce(x, y, fg, interpret=False):
    b, c, h, w = x.shape
    hw = h * w
    rows = b * c
    blk = 512
    nb = rows // blk
    x2 = x.reshape(rows, hw)
    y3 = y.reshape(nb, 1, blk)
    fg3 = fg.reshape(nb, 1, blk)
    out = pl.pallas_call(
        functools.partial(_block_kernel, n_total=rows * hw),
        grid=(nb,),
        in_specs=[
            pl.BlockSpec((blk, hw), lambda i: (i, 0)),
            pl.BlockSpec((1, 1, blk), lambda i: (i, 0, 0)),
            pl.BlockSpec((1, 1, blk), lambda i: (i, 0, 0)),
        ],
        out_specs=pl.BlockSpec((1, 1), lambda i: (0, 0)),
        out_shape=jax.ShapeDtypeStruct((1, 1), jnp.float32),
        interpret=interpret,
    )(x2, y3, fg3)
    return out[0, 0]


def kernel(x, y, fg, iter):
    return _spatial_bce(x, y, fg) + jnp.asarray(iter, jnp.float32) * 0.0


# speedup vs baseline: 1.7541x; 1.0116x over previous
import functools

import jax
import jax.numpy as jnp
from jax.experimental import pallas as pl

_EPS = 1e-08


def _block_kernel(x_ref, y_ref, fg_ref, out_ref, *, n_total):
    i = pl.program_id(0)
    nb = pl.num_programs(0)
    hw = x_ref.shape[1]
    blk = x_ref.shape[0]

    s = jax.nn.sigmoid(x_ref[...])
    si = jax.lax.bitcast_convert_type(s, jnp.int32)

    fg = fg_ref[0, 0, :]
    y = y_ref[0, 0, :]
    kp1 = (fg * hw).astype(jnp.int32) + 1

    def count_ge(data16, cand16b):
        m = (data16 >= cand16b).astype(jnp.int16)
        n = m.shape[1]
        while n > 128:
            n //= 2
            m = m[:, :n] + m[:, n:]
        return jnp.sum(m.astype(jnp.int32), axis=1)

    def search15(data16, target):
        def step(_, carry):
            r, bit = carry
            cand = r | bit
            cnt = count_ge(data16, cand.astype(jnp.int16)[:, None])
            return jnp.where(cnt >= target, cand, r), bit >> 1
        r0 = jnp.zeros((blk,), jnp.int32)
        bit0 = jnp.full((blk,), 1 << 14, jnp.int32)
        r, _ = jax.lax.fori_loop(0, 15, step, (r0, bit0))
        return r

    sh16 = (si >> 15).astype(jnp.int16)
    rh = search15(sh16, kp1)

    rh16b = rh.astype(jnp.int16)[:, None]
    c_gt = count_ge(sh16, rh16b + jnp.int16(1))
    sl16 = jnp.where(sh16 == rh16b,
                     (si & 0x7FFF).astype(jnp.int16), jnp.int16(-1))
    rl = search15(sl16, kp1 - c_gt)

    r = (rh << 15) | rl
    t = jax.lax.bitcast_convert_type(r, jnp.float32)
    t = jnp.maximum(t, 1e-4)[:, None]
    yb = y[:, None]

    u = s * (1.0 / t)
    h_low = u * (2.0 - u)
    one_m_t = 1.0 - t
    alpha = 1.0 / jnp.maximum(one_m_t * one_m_t, _EPS)
    h_high = alpha * (1.0 - s) * (one_m_t + (s - t))
    piece = jnp.where(s <= t, h_low, h_high) * yb
    neg = -(1.0 - yb) * jnp.log(jnp.maximum(1.0 - s, _EPS))
    bsum = jnp.sum(piece + neg)

    prev = jnp.where(i == 0, jnp.zeros((1, 1), jnp.float32), out_ref[...])
    acc = prev + bsum
    out_ref[...] = jnp.where(i == nb - 1, acc / n_total, acc)


@functools.partial(jax.jit, static_argnames=("interpret",))
def _spatial_bce(x, y, fg, interpret=False):
    b, c, h, w = x.shape
    hw = h * w
    rows = b * c
    blk = 640
    nb = rows // blk
    x2 = x.reshape(rows, hw)
    y3 = y.reshape(nb, 1, blk)
    fg3 = fg.reshape(nb, 1, blk)
    out = pl.pallas_call(
        functools.partial(_block_kernel, n_total=rows * hw),
        grid=(nb,),
        in_specs=[
            pl.BlockSpec((blk, hw), lambda i: (i, 0)),
            pl.BlockSpec((1, 1, blk), lambda i: (i, 0, 0)),
            pl.BlockSpec((1, 1, blk), lambda i: (i, 0, 0)),
        ],
        out_specs=pl.BlockSpec((1, 1), lambda i: (0, 0)),
        out_shape=jax.ShapeDtypeStruct((1, 1), jnp.float32),
        interpret=interpret,
    )(x2, y3, fg3)
    return out[0, 0]


def kernel(x, y, fg, iter):
    return _spatial_bce(x, y, fg) + jnp.asarray(iter, jnp.float32) * 0.0
